# baseline (speedup 1.0000x reference)
import functools

import jax
import jax.numpy as jnp
from jax import lax
from jax.experimental import pallas as pl
from jax.experimental.pallas import tpu as pltpu
from jax.experimental.pallas import tpu_sc as plsc

NLIG = 50000
NKP = 5000
V = 16

E_LL = 800000
E_KL = 400000
E_LLP = 802816
E_KLP = 401408
E_BLK = 1568
N_BLK = 2000
K_BLK = 1000

NROW = 50048
_DUMMY = 50040
WG = 16
NG = 7

_NC, _NS, _NW = 2, 16, 32
_C = 128

_S0, _S1 = 0, 64
_X0, _X1 = 112, 115
_ROW = 128


def _gather_body(tab_hbm, idx_hbm, out_hbm, idx_v, rows_v, sem0, sem1,
                 *, perw):
    wid = lax.axis_index("s") * _NC + lax.axis_index("c")
    sems = (sem0, sem1)

    def issue(j, k):
        off = (wid + j * _NW) * _C
        pltpu.sync_copy(idx_hbm.at[pl.ds(off, _C)], idx_v.at[k])
        pltpu.async_copy(tab_hbm.at[idx_v.at[k]], rows_v.at[k], sems[k])

    for k in range(2):
        issue(k, k)

    def gstep(g, carry):
        for k in range(2):
            j = g * 2 + k
            pltpu.make_async_copy(
                tab_hbm.at[idx_v.at[k]], rows_v.at[k], sems[k]).wait()
            off = (wid + j * _NW) * _C
            pltpu.sync_copy(rows_v.at[k], out_hbm.at[pl.ds(off, _C)])
            jn = j + 2

            @pl.when(jn < perw)
            def _():
                issue(jn, k)
        return carry

    lax.fori_loop(0, perw // 2, gstep, 0)


def _sc_gather(table, idx):
    eg = idx.shape[0]
    perw = eg // _C // _NW
    return pl.kernel(
        functools.partial(_gather_body, perw=perw),
        out_type=jax.ShapeDtypeStruct((eg, _ROW), jnp.float32),
        mesh=plsc.VectorSubcoreMesh(core_axis_name="c", subcore_axis_name="s"),
        scratch_types=[
            pltpu.VMEM((2, _C), jnp.int32),
            pltpu.VMEM((2, _C, _ROW), jnp.float32),
            pltpu.SemaphoreType.DMA,
            pltpu.SemaphoreType.DMA,
        ],
    )(table, idx)


_TPW = NROW // _NS


def _scatter_body(mll_hbm, mkl_hbm, dll_hbm, dkl_hbm, out_hbm,
                  idx_v, msg_v, zero_v, acc):
    cid = lax.axis_index("c")
    sid = lax.axis_index("s")
    wid = sid * _NC + cid
    r0 = sid * _TPW

    zv = jnp.zeros((16,), jnp.float32)
    for r in range(_C):
        zero_v[r, pl.ds(0, 16)] = zv

    def stripe_zero():
        for j in range(24):
            pltpu.sync_copy(zero_v, acc.at[pl.ds(r0 + j * _C, _C)])
        pltpu.sync_copy(zero_v.at[pl.ds(0, 56)],
                        acc.at[pl.ds(r0 + 24 * _C, 56)])

    def stripe_dump(gk):
        for j in range(24):
            pltpu.sync_copy(acc.at[pl.ds(r0 + j * _C, _C)],
                            out_hbm.at[gk, cid, pl.ds(r0 + j * _C, _C)])
        pltpu.sync_copy(acc.at[pl.ds(r0 + 24 * _C, 56)],
                        out_hbm.at[gk, cid, pl.ds(r0 + 24 * _C, 56)])

    def scat_loop(msg_hbm, didx_hbm, nchunk_w, gk):
        def step(j, carry):
            off = (wid + j * _NW) * _C
            pltpu.sync_copy(didx_hbm.at[pl.ds(off, _C)], idx_v)
            pltpu.sync_copy(msg_hbm.at[gk, pl.ds(off, _C)], msg_v)
            pltpu.sync_copy(msg_v, acc.at[pl.ds(r0, _C)])
            return carry
        lax.fori_loop(0, nchunk_w, step, 0)

    for gk in range(NG):
        plsc.subcore_barrier()
        stripe_zero()
        plsc.subcore_barrier()
        scat_loop(mll_hbm, dll_hbm, E_LLP // _C // _NW, gk)
        scat_loop(mkl_hbm, dkl_hbm, E_KLP // _C // _NW, gk)
        plsc.subcore_barrier()
        stripe_dump(gk)


def _sc_scatter(mll, mkl, dll, dkl):
    return pl.kernel(
        _scatter_body,
        out_type=jax.ShapeDtypeStruct((NG, _NC, NROW, WG), jnp.float32),
        mesh=plsc.VectorSubcoreMesh(core_axis_name="c", subcore_axis_name="s"),
        scratch_types=[
            pltpu.VMEM((_C,), jnp.int32),
            pltpu.VMEM((_C, WG), jnp.float32),
            pltpu.VMEM((_C, WG), jnp.float32),
            pltpu.VMEM_SHARED((NROW, WG), jnp.float32),
        ],
    )(mll, mkl, dll, dkl)


def _silu(x):
    return x * jax.nn.sigmoid(x)


def _gvp_block(s, mv, wh, wu, wf, b, gate):
    E = s.shape[0]

    def csum(a):
        return a[0:E] + a[E:2 * E] + a[2 * E:3 * E]

    vh = jnp.dot(mv, wh, preferred_element_type=jnp.float32)
    sh = jnp.sqrt(csum(vh * vh) + 1e-8)
    s_out = _silu(jnp.dot(jnp.concatenate([s, sh], axis=1), wf,
                          preferred_element_type=jnp.float32) + b)
    vu = jnp.dot(vh, wu, preferred_element_type=jnp.float32)
    if gate:
        n = jnp.sqrt(csum(vu * vu) + 1e-8)
        g = jax.nn.sigmoid(n)
        vu = jnp.concatenate([g, g, g], axis=0) * vu
    return s_out, vu


def _msg_body(src_ref, dst_ref, w1h, w1u, w1f, b1, w2h, w2u, w2f, b2,
              w3h, w3u, w3f, b3, *gouts):
    src = src_ref[...]
    dst = dst_ref[...]
    E = src.shape[0]
    diff = dst[:, _X0:_X1] - src[:, _X0:_X1]
    nrm = jnp.sqrt(jnp.sum(diff * diff, axis=1, keepdims=True))
    d = diff / (nrm + 1e-8)
    mv = jnp.concatenate([
        jnp.concatenate([src[:, 64:80], dst[:, 64:80], d[:, 0:1]], axis=1),
        jnp.concatenate([src[:, 80:96], dst[:, 80:96], d[:, 1:2]], axis=1),
        jnp.concatenate([src[:, 96:112], dst[:, 96:112], d[:, 2:3]], axis=1),
    ], axis=0)
    s = jnp.concatenate([src[:, _S0:_S1], dst[:, _S0:_S1]], axis=1)
    s, v = _gvp_block(s, mv, w1h[...], w1u[...], w1f[...], b1[...], True)
    s, v = _gvp_block(s, v, w2h[...], w2u[...], w2f[...], b2[...], True)
    s, v = _gvp_block(s, v, w3h[...], w3u[...], w3f[...], b3[...], True)
    (gout,) = gouts
    m = jnp.concatenate([s, v[0:E], v[E:2 * E], v[2 * E:3 * E]], axis=1)
    for g in range(NG):
        gout[g] = m[:, g * WG:(g + 1) * WG]


def _ln(x, g, b):
    mu = jnp.mean(x, axis=-1, keepdims=True)
    var = jnp.mean((x - mu) * (x - mu), axis=-1, keepdims=True)
    return (x - mu) / jnp.sqrt(var + 1e-5) * g + b


def _upd_body(tab_ref, agg_ref, u1h, u1u, u1f, ub1,
              u2h, u2u, u2f, ub2, lng, lnb, out_ref):
    tab = tab_ref[...]
    E = tab.shape[0]
    a = agg_ref[...]
    agg = jnp.concatenate([a[g, 0] + a[g, 1] for g in range(NG)], axis=1)
    s0 = tab[:, _S0:_S1]
    mv = jnp.concatenate([
        jnp.concatenate([tab[:, 64:80], agg[:, 64:80]], axis=1),
        jnp.concatenate([tab[:, 80:96], agg[:, 80:96]], axis=1),
        jnp.concatenate([tab[:, 96:112], agg[:, 96:112]], axis=1),
    ], axis=0)
    s = jnp.concatenate([s0, agg[:, _S0:_S1]], axis=1)
    s, v = _gvp_block(s, mv, u1h[...], u1u[...], u1f[...], ub1[...], True)
    s, v = _gvp_block(s, v, u2h[...], u2u[...], u2f[...], ub2[...], True)
    s_new = _ln(s0 + s, lng[...], lnb[...])
    out_ref[...] = jnp.concatenate(
        [s_new,
         tab[:, 64:80] + v[0:E],
         tab[:, 80:96] + v[E:2 * E],
         tab[:, 96:112] + v[2 * E:3 * E],
         tab[:, _X0:_X1],
         jnp.zeros((E, _ROW - _X1), jnp.float32)], axis=1)


def _enc_body(h0_ref, bidx_ref, x_ref, v_ref, tpad, w, b, lng, lnb, out_ref):
    h0 = h0_ref[...]
    E = h0.shape[0]
    iota = lax.broadcasted_iota(jnp.int32, (E, 256), 1).astype(jnp.float32)
    oh = (bidx_ref[...] == iota).astype(jnp.float32)
    t = jnp.dot(oh, tpad[...], preferred_element_type=jnp.float32,
                precision=jax.lax.Precision.HIGHEST)
    s = _ln(_silu(jnp.dot(jnp.concatenate([h0, t], axis=1), w[...],
                          preferred_element_type=jnp.float32) + b[...]),
            lng[...], lnb[...])
    if v_ref is None:
        vpart = jnp.zeros((E, 48), jnp.float32)
    else:
        vpart = v_ref[...]
    out_ref[...] = jnp.concatenate(
        [s, vpart, x_ref[...], jnp.zeros((E, _ROW - _X1), jnp.float32)],
        axis=1)


def _enc_body_nov(h0_ref, bidx_ref, x_ref, tpad, w, b, lng, lnb, out_ref):
    _enc_body(h0_ref, bidx_ref, x_ref, None, tpad, w, b, lng, lnb, out_ref)


def _noise_body(tab_ref, n1h, n1u, n1f, nb1, n2h, n2u, n2f, nb2,
                n3h, n3u, n3f, nb3, ow, ob, out_ref):
    tab = tab_ref[...]
    E = tab.shape[0]
    s = tab[:, _S0:_S1]
    mv = jnp.concatenate(
        [tab[:, 64:80], tab[:, 80:96], tab[:, 96:112]], axis=0)
    s, v = _gvp_block(s, mv, n1h[...], n1u[...], n1f[...], nb1[...], True)
    s, v = _gvp_block(s, v, n2h[...], n2u[...], n2f[...], nb2[...], True)
    s, v = _gvp_block(s, v, n3h[...], n3u[...], n3f[...], nb3[...], False)
    eps = jnp.dot(s, ow[...], preferred_element_type=jnp.float32) + ob[...]
    out_ref[...] = jnp.concatenate(
        [eps, v[0:E], v[E:2 * E], v[2 * E:3 * E],
         jnp.zeros((E, _ROW - 67), jnp.float32)], axis=1)


def _bcast(shape):
    return pl.BlockSpec(shape, lambda i: (0,) * len(shape))


def _gvp_args(p):
    return (p['Wh'], p['Wu'], p['Wf'], p['bf'].reshape(1, -1))


def _edge_messages(src_arr, src_off, dst_arr, dst_off, nblk, chain):
    ws = _gvp_args(chain[0]) + _gvp_args(chain[1]) + _gvp_args(chain[2])
    eout = nblk * E_BLK
    gshape = jax.ShapeDtypeStruct((NG, eout, WG), jnp.float32)
    return pl.pallas_call(
        _msg_body,
        grid=(nblk,),
        in_specs=[pl.BlockSpec((E_BLK, _ROW),
                               lambda i, o=src_off: (i + o, 0)),
                  pl.BlockSpec((E_BLK, _ROW),
                               lambda i, o=dst_off: (i + o, 0))]
                 + [_bcast(w.shape) for w in ws],
        out_specs=pl.BlockSpec((NG, E_BLK, WG), lambda i: (0, i, 0)),
        out_shape=gshape,
    )(src_arr, dst_arr, *ws)


def _update(tab, aggs, lp):
    ws = (_gvp_args(lp['upd'][0]) + _gvp_args(lp['upd'][1])
          + (lp['ln_g'].reshape(1, -1), lp['ln_b'].reshape(1, -1)))
    return pl.pallas_call(
        _upd_body,
        grid=(NLIG // N_BLK,),
        in_specs=[pl.BlockSpec((N_BLK, _ROW), lambda i: (i, 0))]
                 + [pl.BlockSpec((NG, _NC, N_BLK, WG),
                                 lambda i: (0, 0, i, 0))]
                 + [_bcast(w.shape) for w in ws],
        out_specs=pl.BlockSpec((N_BLK, _ROW), lambda i: (i, 0)),
        out_shape=jax.ShapeDtypeStruct((NLIG, _ROW), jnp.float32),
    )(tab, *aggs, *ws)


def _encode(h0, bidx, x, vcm, tpad, w, b, lng, lnb, blk):
    n = h0.shape[0]
    ws = (w, b.reshape(1, -1), lng.reshape(1, -1), lnb.reshape(1, -1))
    if vcm is not None:
        body = _enc_body
        vspecs = [pl.BlockSpec((blk, 48), lambda i: (i, 0))]
        vargs = (vcm,)
    else:
        body = _enc_body_nov
        vspecs = []
        vargs = ()
    return pl.pallas_call(
        body,
        grid=(n // blk,),
        in_specs=[pl.BlockSpec((blk, 64), lambda i: (i, 0)),
                  pl.BlockSpec((blk, 1), lambda i: (i, 0)),
                  pl.BlockSpec((blk, 3), lambda i: (i, 0))]
                 + vspecs
                 + [_bcast((256, 1))]
                 + [_bcast(x_.shape) for x_ in ws],
        out_specs=pl.BlockSpec((blk, _ROW), lambda i: (i, 0)),
        out_shape=jax.ShapeDtypeStruct((n, _ROW), jnp.float32),
    )(h0, bidx, x, *vargs, tpad, *ws)


def _noise_head(tab, noise, ow, ob):
    ws = (_gvp_args(noise[0]) + _gvp_args(noise[1]) + _gvp_args(noise[2])
          + (ow, ob.reshape(1, -1)))
    return pl.pallas_call(
        _noise_body,
        grid=(NLIG // N_BLK,),
        in_specs=[pl.BlockSpec((N_BLK, _ROW), lambda i: (i, 0))]
                 + [_bcast(w.shape) for w in ws],
        out_specs=pl.BlockSpec((N_BLK, _ROW), lambda i: (i, 0)),
        out_shape=jax.ShapeDtypeStruct((NLIG, _ROW), jnp.float32),
    )(tab, *ws)


def _padi(a, n, fill):
    a = a.astype(jnp.int32)
    return jnp.concatenate(
        [a, jnp.full((n - a.shape[0],), fill, jnp.int32)])


def kernel(lig_h0, lig_x0, kp_h0, kp_x0, kp_v0, timestep, lig_batch_idx,
           kp_batch_idx, ll_edge_index, kl_src, kl_dst, params):
    f32 = jnp.float32
    tpad = jnp.zeros((256, 1), f32).at[:timestep.shape[0], 0].set(timestep)
    bidx_lig = lig_batch_idx.astype(f32)[:, None]
    bidx_kp = kp_batch_idx.astype(f32)[:, None]
    kp_vcm = jnp.transpose(kp_v0, (0, 2, 1)).reshape(NKP, 48)

    lig_tab = _encode(lig_h0, bidx_lig, lig_x0, None, tpad,
                      params['lig_enc_W'], params['lig_enc_b'],
                      params['lig_ln_g'], params['lig_ln_b'], N_BLK)
    kp_tab = _encode(kp_h0, bidx_kp, kp_x0, kp_vcm, tpad,
                     params['kp_enc_W'], params['kp_enc_b'],
                     params['kp_ln_g'], params['kp_ln_b'], K_BLK)

    src = ll_edge_index[0]
    dst = ll_edge_index[1]
    gidx = jnp.concatenate([_padi(src, E_LLP, 0), _padi(dst, E_LLP, 0),
                            _padi(kl_dst, E_KLP, 0)])
    dll_s = _padi(dst, E_LLP, _DUMMY)
    dkl_s = _padi(kl_dst, E_KLP, _DUMMY)

    kp_rows = _sc_gather(kp_tab, _padi(kl_src, E_KLP, 0))

    nb_ll = E_LLP // E_BLK
    nb_kl = E_KLP // E_BLK
    for lp in params['convs']:
        rows = _sc_gather(lig_tab, gidx)
        mll = _edge_messages(rows, 0, rows, nb_ll, nb_ll, lp['ll_msg'])
        mkl = _edge_messages(kp_rows, 0, rows, 2 * nb_ll, nb_kl,
                             lp['kl_msg'])
        aggs = jnp.stack([jnp.stack([
            jax.ops.segment_sum(mll[g], dll_s, num_segments=NROW)
            + jax.ops.segment_sum(mkl[g], dkl_s, num_segments=NROW),
            jnp.zeros((NROW, WG), jnp.float32)]) for g in range(NG)])
        lig_tab = _update(lig_tab, (aggs,), lp)

    out = _noise_head(lig_tab, params['noise'], params['out_W'],
                      params['out_b'])
    eps_h = out[:, 0:64]
    v = out[:, 64:67].reshape(NLIG, 1, 3)
    return eps_h, v


# speedup vs baseline: 1.8077x; 1.8077x over previous
import functools

import jax
import jax.numpy as jnp
from jax.experimental import pallas as pl
from jax.experimental.pallas import tpu as pltpu

NLIG = 50000
NKP = 5000
H = 64
V = 16

E_BLK = 1600
N_BLK = 2000
K_BLK = 1000

_S0, _S1 = 0, 64
_V0, _V1 = 64, 112
_X0, _X1 = 112, 115
_ROW = 128


def _silu(x):
    return x * jax.nn.sigmoid(x)


def _gvp_block(s, mv, wh, wu, wf, b, gate):
    E = s.shape[0]

    def csum(a):
        return a[0:E] + a[E:2 * E] + a[2 * E:3 * E]

    vh = jnp.dot(mv, wh, preferred_element_type=jnp.float32)
    sh = jnp.sqrt(csum(vh * vh) + 1e-8)
    s_out = _silu(jnp.dot(jnp.concatenate([s, sh], axis=1), wf,
                          preferred_element_type=jnp.float32) + b)
    vu = jnp.dot(vh, wu, preferred_element_type=jnp.float32)
    if gate:
        n = jnp.sqrt(csum(vu * vu) + 1e-8)
        g = jax.nn.sigmoid(n)
        vu = jnp.concatenate([g, g, g], axis=0) * vu
    return s_out, vu


def _msg_body(src_ref, dst_ref, w1h, w1u, w1f, b1, w2h, w2u, w2f, b2,
              w3h, w3u, w3f, b3, out_ref):
    src = src_ref[...]
    dst = dst_ref[...]
    E = src.shape[0]
    diff = dst[:, _X0:_X1] - src[:, _X0:_X1]
    nrm = jnp.sqrt(jnp.sum(diff * diff, axis=1, keepdims=True))
    d = diff / (nrm + 1e-8)
    mv = jnp.concatenate([
        jnp.concatenate([src[:, 64:80], dst[:, 64:80], d[:, 0:1]], axis=1),
        jnp.concatenate([src[:, 80:96], dst[:, 80:96], d[:, 1:2]], axis=1),
        jnp.concatenate([src[:, 96:112], dst[:, 96:112], d[:, 2:3]], axis=1),
    ], axis=0)
    s = jnp.concatenate([src[:, _S0:_S1], dst[:, _S0:_S1]], axis=1)
    s, v = _gvp_block(s, mv, w1h[...], w1u[...], w1f[...], b1[...], True)
    s, v = _gvp_block(s, v, w2h[...], w2u[...], w2f[...], b2[...], True)
    s, v = _gvp_block(s, v, w3h[...], w3u[...], w3f[...], b3[...], True)
    out_ref[...] = jnp.concatenate(
        [s, v[0:E], v[E:2 * E], v[2 * E:3 * E],
         jnp.zeros((E, _ROW - _V1), jnp.float32)], axis=1)


def _ln(x, g, b):
    mu = jnp.mean(x, axis=-1, keepdims=True)
    var = jnp.mean((x - mu) * (x - mu), axis=-1, keepdims=True)
    return (x - mu) / jnp.sqrt(var + 1e-5) * g + b


def _upd_body(tab_ref, agg_ref, u1h, u1u, u1f, ub1, u2h, u2u, u2f, ub2,
              lng, lnb, out_ref):
    tab = tab_ref[...]
    agg = agg_ref[...]
    E = tab.shape[0]
    s0 = tab[:, _S0:_S1]
    mv = jnp.concatenate([
        jnp.concatenate([tab[:, 64:80], agg[:, 64:80]], axis=1),
        jnp.concatenate([tab[:, 80:96], agg[:, 80:96]], axis=1),
        jnp.concatenate([tab[:, 96:112], agg[:, 96:112]], axis=1),
    ], axis=0)
    s = jnp.concatenate([s0, agg[:, _S0:_S1]], axis=1)
    s, v = _gvp_block(s, mv, u1h[...], u1u[...], u1f[...], ub1[...], True)
    s, v = _gvp_block(s, v, u2h[...], u2u[...], u2f[...], ub2[...], True)
    s_new = _ln(s0 + s, lng[...], lnb[...])
    out_ref[...] = jnp.concatenate(
        [s_new,
         tab[:, 64:80] + v[0:E],
         tab[:, 80:96] + v[E:2 * E],
         tab[:, 96:112] + v[2 * E:3 * E],
         tab[:, _X0:_X1],
         jnp.zeros((E, _ROW - _X1), jnp.float32)], axis=1)


def _enc_body(inp_ref, w, b, lng, lnb, out_ref, *, has_v):
    inp = inp_ref[...]
    E = inp.shape[0]
    s = _ln(_silu(jnp.dot(inp[:, 0:65], w[...],
                          preferred_element_type=jnp.float32) + b[...]),
            lng[...], lnb[...])
    if has_v:
        vpart = inp[:, 68:116]
    else:
        vpart = jnp.zeros((E, 48), jnp.float32)
    out_ref[...] = jnp.concatenate(
        [s, vpart, inp[:, 65:68], jnp.zeros((E, _ROW - _X1), jnp.float32)],
        axis=1)


def _noise_body(tab_ref, n1h, n1u, n1f, nb1, n2h, n2u, n2f, nb2,
                n3h, n3u, n3f, nb3, ow, ob, out_ref):
    tab = tab_ref[...]
    E = tab.shape[0]
    s = tab[:, _S0:_S1]
    mv = jnp.concatenate(
        [tab[:, 64:80], tab[:, 80:96], tab[:, 96:112]], axis=0)
    s, v = _gvp_block(s, mv, n1h[...], n1u[...], n1f[...], nb1[...], True)
    s, v = _gvp_block(s, v, n2h[...], n2u[...], n2f[...], nb2[...], True)
    s, v = _gvp_block(s, v, n3h[...], n3u[...], n3f[...], nb3[...], False)
    eps = jnp.dot(s, ow[...], preferred_element_type=jnp.float32) + ob[...]
    out_ref[...] = jnp.concatenate(
        [eps, v[0:E], v[E:2 * E], v[2 * E:3 * E],
         jnp.zeros((E, _ROW - 67), jnp.float32)], axis=1)


def _bcast(shape):
    return pl.BlockSpec(shape, lambda i: (0,) * len(shape))


def _gvp_args(p):
    return (p['Wh'], p['Wu'], p['Wf'], p['bf'].reshape(1, -1))


def _edge_messages(src_rows, dst_rows, chain):
    E = src_rows.shape[0]
    ws = _gvp_args(chain[0]) + _gvp_args(chain[1]) + _gvp_args(chain[2])
    return pl.pallas_call(
        _msg_body,
        grid=(E // E_BLK,),
        in_specs=[pl.BlockSpec((E_BLK, _ROW), lambda i: (i, 0)),
                  pl.BlockSpec((E_BLK, _ROW), lambda i: (i, 0))]
                 + [_bcast(w.shape) for w in ws],
        out_specs=pl.BlockSpec((E_BLK, _ROW), lambda i: (i, 0)),
        out_shape=jax.ShapeDtypeStruct((E, _ROW), jnp.float32),
    )(src_rows, dst_rows, *ws)


def _update(tab, agg, lp):
    ws = (_gvp_args(lp['upd'][0]) + _gvp_args(lp['upd'][1])
          + (lp['ln_g'].reshape(1, -1), lp['ln_b'].reshape(1, -1)))
    return pl.pallas_call(
        _upd_body,
        grid=(NLIG // N_BLK,),
        in_specs=[pl.BlockSpec((N_BLK, _ROW), lambda i: (i, 0)),
                  pl.BlockSpec((N_BLK, _ROW), lambda i: (i, 0))]
                 + [_bcast(w.shape) for w in ws],
        out_specs=pl.BlockSpec((N_BLK, _ROW), lambda i: (i, 0)),
        out_shape=jax.ShapeDtypeStruct((NLIG, _ROW), jnp.float32),
    )(tab, agg, *ws)


def _encode(inp, w, b, lng, lnb, blk, has_v):
    n, c = inp.shape
    ws = (w, b.reshape(1, -1), lng.reshape(1, -1), lnb.reshape(1, -1))
    return pl.pallas_call(
        functools.partial(_enc_body, has_v=has_v),
        grid=(n // blk,),
        in_specs=[pl.BlockSpec((blk, c), lambda i: (i, 0))]
                 + [_bcast(x.shape) for x in ws],
        out_specs=pl.BlockSpec((blk, _ROW), lambda i: (i, 0)),
        out_shape=jax.ShapeDtypeStruct((n, _ROW), jnp.float32),
    )(inp, *ws)


def _noise_head(tab, noise, ow, ob):
    ws = (_gvp_args(noise[0]) + _gvp_args(noise[1]) + _gvp_args(noise[2])
          + (ow, ob.reshape(1, -1)))
    return pl.pallas_call(
        _noise_body,
        grid=(NLIG // N_BLK,),
        in_specs=[pl.BlockSpec((N_BLK, _ROW), lambda i: (i, 0))]
                 + [_bcast(w.shape) for w in ws],
        out_specs=pl.BlockSpec((N_BLK, _ROW), lambda i: (i, 0)),
        out_shape=jax.ShapeDtypeStruct((NLIG, _ROW), jnp.float32),
    )(tab, *ws)


def kernel(lig_h0, lig_x0, kp_h0, kp_x0, kp_v0, timestep, lig_batch_idx,
           kp_batch_idx, ll_edge_index, kl_src, kl_dst, params):
    f32 = jnp.float32
    t_lig = timestep[lig_batch_idx][:, None].astype(f32)
    t_kp = timestep[kp_batch_idx][:, None].astype(f32)
    enc_lig = jnp.concatenate(
        [lig_h0, t_lig, lig_x0, jnp.zeros((NLIG, 4), f32)], axis=1)
    kp_vcm = jnp.transpose(kp_v0, (0, 2, 1)).reshape(NKP, 48)
    enc_kp = jnp.concatenate([kp_h0, t_kp, kp_x0, kp_vcm], axis=1)

    lig_tab = _encode(enc_lig, params['lig_enc_W'], params['lig_enc_b'],
                      params['lig_ln_g'], params['lig_ln_b'], N_BLK, False)
    kp_tab = _encode(enc_kp, params['kp_enc_W'], params['kp_enc_b'],
                     params['kp_ln_g'], params['kp_ln_b'], K_BLK, True)

    src = ll_edge_index[0]
    dst = ll_edge_index[1]
    for lp in params['convs']:
        msg_ll = _edge_messages(jnp.take(lig_tab, src, axis=0),
                                jnp.take(lig_tab, dst, axis=0), lp['ll_msg'])
        msg_kl = _edge_messages(jnp.take(kp_tab, kl_src, axis=0),
                                jnp.take(lig_tab, kl_dst, axis=0),
                                lp['kl_msg'])
        agg = (jax.ops.segment_sum(msg_ll, dst, num_segments=NLIG)
               + jax.ops.segment_sum(msg_kl, kl_dst, num_segments=NLIG))
        lig_tab = _update(lig_tab, agg, lp)

    out = _noise_head(lig_tab, params['noise'], params['out_W'],
                      params['out_b'])
    eps_h = out[:, 0:64]
    v = out[:, 64:67].reshape(NLIG, 1, 3)
    return eps_h, v
